# baseline (speedup 1.0000x reference)
import functools
import math

import jax
import jax.numpy as jnp
from jax import lax
from jax.experimental import pallas as pl
from jax.experimental.pallas import tpu as pltpu
from jax.experimental.pallas import tpu_sc as plsc

_NC = 2
_NS = 16
_NW = _NC * _NS
_L = 16

_GATHER_W = 128
_CHUNK = 512
_K = _CHUNK // _GATHER_W

_SCALE = math.sqrt(64.0)


@functools.partial(jax.jit, static_argnums=(2, 3))
def _embed_call(x_rows, table, n_rows, d):
    n_chunks = n_rows // (_NW * _CHUNK)
    rows_per_w = n_rows // _NW

    mesh = plsc.VectorSubcoreMesh(
        core_axis_name="c", subcore_axis_name="s",
        num_cores=_NC, num_subcores=_NS)

    idx_rows_per_w = rows_per_w // _GATHER_W

    @functools.partial(
        pl.kernel,
        out_type=jax.ShapeDtypeStruct((n_rows, d), jnp.float32),
        mesh=mesh,
        scratch_types=[
            pltpu.VMEM((idx_rows_per_w, _GATHER_W), jnp.int32),
            pltpu.VMEM((_CHUNK, d), jnp.float32),
            pltpu.SemaphoreType.DMA,
        ],
        compiler_params=pltpu.CompilerParams(use_tc_tiling_on_sc=False),
    )
    def _embed(idx_hbm, table_hbm, out_hbm, idx_v, rows_v, sem):
        wid = lax.axis_index("s") * _NC + lax.axis_index("c")
        row_base = wid * rows_per_w
        idx_row_base = pl.multiple_of(row_base // _GATHER_W, 8)

        pltpu.sync_copy(
            idx_hbm.at[pl.ds(idx_row_base, idx_rows_per_w)], idx_v)

        def chunk_body(g, carry):
            off = row_base + g * _CHUNK
            copies = []
            for j in range(_K):
                copies.append(pltpu.async_copy(
                    table_hbm.at[idx_v.at[g * _K + j]],
                    rows_v.at[pl.ds(j * _GATHER_W, _GATHER_W)],
                    sem))
            for c in copies:
                c.wait()

            def scale_body(r, c2):
                for cc in range(d // _L):
                    sl = pl.ds(cc * _L, _L)
                    rows_v[r, sl] = rows_v[r, sl] * _SCALE
                return c2

            lax.fori_loop(0, _CHUNK, scale_body, 0, unroll=4)

            pltpu.sync_copy(rows_v, out_hbm.at[pl.ds(off, _CHUNK)])
            return carry

        lax.fori_loop(0, n_chunks, chunk_body, 0)

    idx2d = x_rows.reshape(-1, _GATHER_W)
    return _embed(idx2d, table)


def kernel(x, table):
    n_rows = x.shape[0] * x.shape[1]
    d = table.shape[1]
    flat = x.reshape(-1).astype(jnp.int32)
    out = _embed_call(flat, table, n_rows, d)
    return out.reshape(x.shape[0], x.shape[1], d)
